# Initial kernel scaffold; baseline (speedup 1.0000x reference)
import jax
import jax.numpy as jnp
from jax.experimental import pallas as pl


def kernel(x, alpha):
    raise NotImplementedError("write your pallas kernel here")


# speedup vs baseline: 11.1619x; 11.1619x over previous
import jax
import jax.numpy as jnp
from jax import lax
from jax.experimental import pallas as pl
from jax.experimental.pallas import tpu as pltpu

_N = 2048
_D = 256
_K = 33
_KSEL = _K + 1
_MAX_ITER = 3
_ETA = 0.01
_EPS = 1e-8
_BR = 256
_NB = _N // _BR


def _prep_kernel(x_ref, xn_ref, d2_ref):
    x = x_ref[...]
    mean = jnp.mean(x, axis=0, keepdims=True)
    xc = x - mean
    var = jnp.sum(xc * xc, axis=0, keepdims=True) * (1.0 / (_N - 1))
    std = jnp.sqrt(var)
    xn = xc / (std + _EPS)
    xn_ref[...] = xn

    sq_all = jnp.sum(xn * xn, axis=1)[None, :]

    def body(rb, _):
        xr = xn_ref[pl.ds(rb * _BR, _BR), :]
        g = lax.dot_general(xr, xn, (((1,), (1,)), ((), ())),
                            preferred_element_type=jnp.float32)
        sqr = jnp.sum(xr * xr, axis=1, keepdims=True)
        d2 = jnp.maximum(sqr + sq_all - 2.0 * g, 0.0)
        d2_ref[pl.ds(rb * _BR, _BR), :] = d2
        return 0

    lax.fori_loop(0, _NB, body, 0)


def _thresh_kernel(d2_ref, thr_ref):
    def body(rb, _):
        bits = lax.bitcast_convert_type(
            d2_ref[pl.ds(rb * _BR, _BR), :], jnp.int32)
        lo = jnp.zeros((_BR, 1), jnp.int32)
        hi = jnp.full((_BR, 1), jnp.int32(2**31 - 1))

        def bs(i, carry):
            lo, hi = carry
            mid = lo + lax.shift_right_logical(hi - lo, 1)
            cnt = jnp.sum((bits <= mid).astype(jnp.int32), axis=1,
                          keepdims=True)
            take = cnt >= _KSEL
            hi = jnp.where(take, mid, hi)
            lo = jnp.where(take, lo, mid + 1)
            return lo, hi

        lo, hi = lax.fori_loop(0, 31, bs, (lo, hi))
        thr_ref[pl.ds(rb * _BR, _BR), :] = lax.bitcast_convert_type(
            hi, jnp.float32)
        return 0

    lax.fori_loop(0, _NB, body, 0)


def _iter_kernel(xn_ref, d2_ref, thr_ref, alpha_ref, y_out_ref,
                 y_scr, ynext_scr, sq_scr):
    alpha = alpha_ref[0, 0]
    y_scr[...] = xn_ref[...]

    for _t in range(_MAX_ITER):
        def sq_body(rb, _):
            yr = y_scr[pl.ds(rb * _BR, _BR), :]
            sq_scr[0, pl.ds(rb * _BR, _BR)] = jnp.sum(yr * yr, axis=1)
            return 0

        lax.fori_loop(0, _NB, sq_body, 0)

        def blk_body(rb, _):
            yr = y_scr[pl.ds(rb * _BR, _BR), :]
            y_full = y_scr[...]
            g = lax.dot_general(yr, y_full, (((1,), (1,)), ((), ())),
                                preferred_element_type=jnp.float32)
            sq_row = sq_scr[0, :][None, :]
            logits = alpha * (2.0 * g - sq_row)
            d2r = d2_ref[pl.ds(rb * _BR, _BR), :]
            thr = thr_ref[pl.ds(rb * _BR, _BR), :]
            cols = lax.broadcasted_iota(jnp.int32, (_BR, _N), 1)
            rows = rb * _BR + lax.broadcasted_iota(jnp.int32, (_BR, _N), 0)
            mask = (d2r <= thr) & (cols != rows)
            ml = jnp.where(mask, logits, -1e30)
            m = jnp.max(ml, axis=1, keepdims=True)
            e = jnp.where(mask, jnp.exp(ml - m), 0.0)
            s = jnp.sum(e, axis=1, keepdims=True)
            w = e / s
            wy = lax.dot_general(w, y_full, (((1,), (0,)), ((), ())),
                                 preferred_element_type=jnp.float32)
            ynext_scr[pl.ds(rb * _BR, _BR), :] = yr + _ETA * (wy - yr)
            return 0

        lax.fori_loop(0, _NB, blk_body, 0)
        y_scr[...] = ynext_scr[...]

    y_out_ref[...] = y_scr[...]


def kernel(x, alpha):
    xn, d2 = pl.pallas_call(
        _prep_kernel,
        out_shape=[
            jax.ShapeDtypeStruct((_N, _D), jnp.float32),
            jax.ShapeDtypeStruct((_N, _N), jnp.float32),
        ],
    )(x)

    thr = pl.pallas_call(
        _thresh_kernel,
        out_shape=jax.ShapeDtypeStruct((_N, 1), jnp.float32),
    )(d2)

    alpha2d = jnp.asarray(alpha, jnp.float32).reshape(1, 1)
    y = pl.pallas_call(
        _iter_kernel,
        in_specs=[
            pl.BlockSpec(memory_space=pltpu.VMEM),
            pl.BlockSpec(memory_space=pltpu.VMEM),
            pl.BlockSpec(memory_space=pltpu.VMEM),
            pl.BlockSpec(memory_space=pltpu.SMEM),
        ],
        out_shape=jax.ShapeDtypeStruct((_N, _D), jnp.float32),
        scratch_shapes=[
            pltpu.VMEM((_N, _D), jnp.float32),
            pltpu.VMEM((_N, _D), jnp.float32),
            pltpu.VMEM((1, _N), jnp.float32),
        ],
    )(xn, d2, thr, alpha2d)
    return y
